# baseline (speedup 1.0000x reference)
import functools

import jax
import jax.numpy as jnp
from jax import lax
from jax.experimental import pallas as pl
from jax.experimental.pallas import tpu as pltpu

_THRESH = 0.6
_OBJECT_SCALE = 5.0
_NOOBJECT_SCALE = 1.0


def _region_loss_body(out_a0, out_a1, tgt_ref, anc_ref, loss_ref, acc_ref, *, nB, nA, nH, nW, b_blk):
    i = pl.program_id(0)

    t = tgt_ref[0]
    gt_x = (t[:, 0:1] * nW)[:, :, None]
    gt_y = (t[:, 1:2] * nH)[:, :, None]
    gt_w = (t[:, 2:3] * nW)[:, :, None]
    gt_h = (t[:, 3:4] * nH)[:, :, None]
    scale = 2.0 - (t[:, 2:3] * t[:, 3:4])[:, :, None]

    aw0 = anc_ref[0, 0]
    ah0 = anc_ref[0, 1]
    aw1 = anc_ref[1, 0]
    ah1 = anc_ref[1, 1]

    inter0 = jnp.minimum(gt_w, aw0) * jnp.minimum(gt_h, ah0)
    union0 = gt_w * gt_h + 1e-16 + aw0 * ah0 - inter0
    inter1 = jnp.minimum(gt_w, aw1) * jnp.minimum(gt_h, ah1)
    union1 = gt_w * gt_h + 1e-16 + aw1 * ah1 - inter1
    best_is_1 = inter1 * union0 > inter0 * union1

    ix = lax.broadcasted_iota(jnp.int32, (b_blk, nH, nW), 2).astype(jnp.float32)
    iy = lax.broadcasted_iota(jnp.int32, (b_blk, nH, nW), 1).astype(jnp.float32)

    bx1 = gt_x - gt_w * 0.5
    bx2 = gt_x + gt_w * 0.5
    by1 = gt_y - gt_h * 0.5
    by2 = gt_y + gt_h * 0.5
    barea = gt_w * gt_h

    sum_nm = jnp.float32(0.0)
    sum_cn = jnp.float32(0.0)
    for a in range(nA):
        aw = aw1 if a == 1 else aw0
        ah = ah1 if a == 1 else ah0
        out_ref = out_a1 if a == 1 else out_a0
        o0 = out_ref[:, 0, :, :]
        o1 = out_ref[:, 1, :, :]
        o2 = out_ref[:, 2, :, :]
        o3 = out_ref[:, 3, :, :]
        o4 = out_ref[:, 4, :, :]
        x = jax.nn.sigmoid(o0)
        y = jax.nn.sigmoid(o1)
        conf = jax.nn.sigmoid(o4)
        pw = jnp.exp(o2) * aw
        ph = jnp.exp(o3) * ah
        px = x + ix
        py = y + iy
        pwh = pw * 0.5
        phh = ph * 0.5
        cw = jnp.minimum(px + pwh, bx2) - jnp.maximum(px - pwh, bx1)
        ch = jnp.minimum(py + phh, by2) - jnp.maximum(py - phh, by1)
        carea = cw * ch
        uarea = pw * ph + barea - carea
        hot = (jnp.minimum(cw, ch) > 0) & (carea > _THRESH * uarea)
        nmf = jnp.where(hot, 0.0, 1.0)
        sum_nm = sum_nm + jnp.sum(nmf)
        sum_cn = sum_cn + jnp.sum(conf * conf * nmf)

    lane = lax.broadcasted_iota(jnp.int32, (1, nW), 1)
    cols = [[] for _ in range(5)]
    for b in range(b_blk):
        t0 = tgt_ref[0, b, 0]
        t1 = tgt_ref[0, b, 1]
        t2 = tgt_ref[0, b, 2]
        t3 = tgt_ref[0, b, 3]
        s_gt_w = t2 * nW
        s_gt_h = t3 * nH
        s_i0 = jnp.minimum(s_gt_w, aw0) * jnp.minimum(s_gt_h, ah0)
        s_u0 = s_gt_w * s_gt_h + 1e-16 + aw0 * ah0 - s_i0
        s_i1 = jnp.minimum(s_gt_w, aw1) * jnp.minimum(s_gt_h, ah1)
        s_u1 = s_gt_w * s_gt_h + 1e-16 + aw1 * ah1 - s_i1
        bb = jnp.where(s_i1 * s_u0 > s_i0 * s_u1, 1.0, 0.0)
        gi = (t0 * nW).astype(jnp.int32)
        gj = (t1 * nH).astype(jnp.int32)
        ihot = jnp.where(lane == gi, 1.0, 0.0)
        for c in range(5):
            r0 = out_a0[b, c, pl.ds(gj, 1), :]
            r1 = out_a1[b, c, pl.ds(gj, 1), :]
            row = r0 * (1.0 - bb) + r1 * bb
            cols[c].append(jnp.sum(row * ihot).reshape(1, 1))

    g0, g1, g2, g3, g4 = (jnp.concatenate(col, axis=0)[:, :, None] for col in cols)
    xo = jax.nn.sigmoid(g0)
    yo = jax.nn.sigmoid(g1)
    wo = g2
    ho = g3
    co = jax.nn.sigmoid(g4)

    gi_f = jnp.floor(gt_x)
    gj_f = jnp.floor(gt_y)
    tx = gt_x - gi_f
    ty = gt_y - gj_f
    best_aw = jnp.where(best_is_1, aw1, aw0)
    best_ah = jnp.where(best_is_1, ah1, ah0)
    tw = jnp.log(gt_w / best_aw + 1e-16)
    th = jnp.log(gt_h / best_ah + 1e-16)

    pw_o = jnp.exp(wo) * best_aw
    ph_o = jnp.exp(ho) * best_ah
    px_o = xo + gi_f
    py_o = yo + gj_f
    cw_o = jnp.minimum(px_o + pw_o * 0.5, bx2) - jnp.maximum(px_o - pw_o * 0.5, bx1)
    ch_o = jnp.minimum(py_o + ph_o * 0.5, by2) - jnp.maximum(py_o - ph_o * 0.5, by1)
    carea_o = cw_o * ch_o
    uarea_o = pw_o * ph_o + barea - carea_o
    hot_o = (jnp.minimum(cw_o, ch_o) > 0) & (carea_o > _THRESH * uarea_o)
    nm_o = jnp.where(hot_o, 0.0, 1.0)
    sub_nm = jnp.sum(nm_o)
    sub_cn = jnp.sum(nm_o * co * co)

    s2 = scale * scale
    obj = ((xo - tx) ** 2 + (yo - ty) ** 2 + (wo - tw) ** 2 + (ho - th) ** 2) * s2
    obj = obj + _OBJECT_SCALE * (co - 1.0) ** 2
    part_obj = jnp.sum(obj) / jnp.float32(nB)

    @pl.when(i == 0)
    def _init():
        acc_ref[0] = 0.0
        acc_ref[1] = 0.0
        acc_ref[2] = 0.0

    acc_ref[0] = acc_ref[0] + part_obj
    acc_ref[1] = acc_ref[1] + (sum_nm - sub_nm)
    acc_ref[2] = acc_ref[2] + (sum_cn - sub_cn)

    @pl.when(i == pl.num_programs(0) - 1)
    def _fin():
        loss_ref[0, 0] = acc_ref[0] + _NOOBJECT_SCALE * acc_ref[2] / acc_ref[1]


def kernel(output, target, anchors):
    nB, nC, nH, nW = output.shape
    nA = anchors.shape[0]
    b_blk = 16
    grid = (nB // b_blk,)
    body = functools.partial(_region_loss_body, nB=nB, nA=nA, nH=nH, nW=nW, b_blk=b_blk)
    loss = pl.pallas_call(
        body,
        grid=grid,
        in_specs=[
            pl.BlockSpec((b_blk, nC // 2, nH, nW), lambda i: (i, 0, 0, 0)),
            pl.BlockSpec((b_blk, nC // 2, nH, nW), lambda i: (i, 1, 0, 0)),
            pl.BlockSpec((1, b_blk, 4), lambda i: (i, 0, 0)),
            pl.BlockSpec((nA, 2), lambda i: (0, 0)),
        ],
        out_specs=pl.BlockSpec(memory_space=pltpu.SMEM),
        out_shape=jax.ShapeDtypeStruct((1, 1), jnp.float32),
        scratch_shapes=[pltpu.SMEM((3,), jnp.float32)],
    )(output, output, target.reshape(nB // b_blk, b_blk, 4), anchors)
    return loss[0, 0]


# speedup vs baseline: 1.1468x; 1.1468x over previous
import functools

import jax
import jax.numpy as jnp
from jax import lax
from jax.experimental import pallas as pl
from jax.experimental.pallas import tpu as pltpu

_THRESH = 0.6
_OBJECT_SCALE = 5.0
_NOOBJECT_SCALE = 1.0


def _region_loss_body(out_a0, out_a1, tgt_ref, anc_ref, loss_ref, acc_ref, *, nB, nA, nH, nW, b_blk):
    i = pl.program_id(0)

    t = tgt_ref[0]
    gt_x = (t[:, 0:1] * nW)[:, :, None]
    gt_y = (t[:, 1:2] * nH)[:, :, None]
    gt_w = (t[:, 2:3] * nW)[:, :, None]
    gt_h = (t[:, 3:4] * nH)[:, :, None]
    scale = 2.0 - (t[:, 2:3] * t[:, 3:4])[:, :, None]

    aw0 = anc_ref[0, 0]
    ah0 = anc_ref[0, 1]
    aw1 = anc_ref[1, 0]
    ah1 = anc_ref[1, 1]

    gi_f = jnp.floor(gt_x)
    gj_f = jnp.floor(gt_y)
    tx = gt_x - gi_f
    ty = gt_y - gj_f

    inter0 = jnp.minimum(gt_w, aw0) * jnp.minimum(gt_h, ah0)
    union0 = gt_w * gt_h + 1e-16 + aw0 * ah0 - inter0
    inter1 = jnp.minimum(gt_w, aw1) * jnp.minimum(gt_h, ah1)
    union1 = gt_w * gt_h + 1e-16 + aw1 * ah1 - inter1
    best_is_1 = inter1 * union0 > inter0 * union1
    best_aw = jnp.where(best_is_1, aw1, aw0)
    best_ah = jnp.where(best_is_1, ah1, ah0)
    tw = jnp.log(gt_w / best_aw + 1e-16)
    th = jnp.log(gt_h / best_ah + 1e-16)

    ix = lax.broadcasted_iota(jnp.int32, (b_blk, nH, nW), 2).astype(jnp.float32)
    iy = lax.broadcasted_iota(jnp.int32, (b_blk, nH, nW), 1).astype(jnp.float32)

    bx1 = gt_x - gt_w * 0.5
    bx2 = gt_x + gt_w * 0.5
    by1 = gt_y - gt_h * 0.5
    by2 = gt_y + gt_h * 0.5
    barea = gt_w * gt_h

    sum_nm = jnp.float32(0.0)
    sum_cn = jnp.float32(0.0)
    xo = jnp.zeros((b_blk, 1, 1), jnp.float32)
    yo = jnp.zeros((b_blk, 1, 1), jnp.float32)
    wo = jnp.zeros((b_blk, 1, 1), jnp.float32)
    ho = jnp.zeros((b_blk, 1, 1), jnp.float32)
    co = jnp.zeros((b_blk, 1, 1), jnp.float32)

    for a in range(nA):
        aw = aw1 if a == 1 else aw0
        ah = ah1 if a == 1 else ah0
        out_ref = out_a1 if a == 1 else out_a0
        o0 = out_ref[:, 0, :, :]
        o1 = out_ref[:, 1, :, :]
        o2 = out_ref[:, 2, :, :]
        o3 = out_ref[:, 3, :, :]
        o4 = out_ref[:, 4, :, :]
        x = jax.nn.sigmoid(o0)
        y = jax.nn.sigmoid(o1)
        conf = jax.nn.sigmoid(o4)
        pw = jnp.exp(o2) * aw
        ph = jnp.exp(o3) * ah
        px = x + ix
        py = y + iy
        pwh = pw * 0.5
        phh = ph * 0.5
        cw = jnp.minimum(px + pwh, bx2) - jnp.maximum(px - pwh, bx1)
        ch = jnp.minimum(py + phh, by2) - jnp.maximum(py - phh, by1)
        carea = cw * ch
        uarea = pw * ph + barea - carea
        hot = (jnp.minimum(cw, ch) > 0) & (carea > _THRESH * uarea)

        is_best = best_is_1 if a == 1 else ~best_is_1
        onehot = (iy == gj_f) & (ix == gi_f) & is_best
        nmf = jnp.where(hot | onehot, 0.0, 1.0)
        sum_nm = sum_nm + jnp.sum(nmf)
        sum_cn = sum_cn + jnp.sum(conf * conf * nmf)

        ohf = jnp.where(onehot, 1.0, 0.0)
        xo = xo + jnp.sum(x * ohf, axis=(1, 2), keepdims=True)
        yo = yo + jnp.sum(y * ohf, axis=(1, 2), keepdims=True)
        wo = wo + jnp.sum(o2 * ohf, axis=(1, 2), keepdims=True)
        ho = ho + jnp.sum(o3 * ohf, axis=(1, 2), keepdims=True)
        co = co + jnp.sum(conf * ohf, axis=(1, 2), keepdims=True)

    s2 = scale * scale
    obj = ((xo - tx) ** 2 + (yo - ty) ** 2 + (wo - tw) ** 2 + (ho - th) ** 2) * s2
    obj = obj + _OBJECT_SCALE * (co - 1.0) ** 2
    part_obj = jnp.sum(obj) / jnp.float32(nB)

    @pl.when(i == 0)
    def _init():
        acc_ref[0] = 0.0
        acc_ref[1] = 0.0
        acc_ref[2] = 0.0

    acc_ref[0] = acc_ref[0] + part_obj
    acc_ref[1] = acc_ref[1] + sum_nm
    acc_ref[2] = acc_ref[2] + sum_cn

    @pl.when(i == pl.num_programs(0) - 1)
    def _fin():
        loss_ref[0, 0] = acc_ref[0] + _NOOBJECT_SCALE * acc_ref[2] / acc_ref[1]


def kernel(output, target, anchors):
    nB, nC, nH, nW = output.shape
    nA = anchors.shape[0]
    b_blk = 16
    grid = (nB // b_blk,)
    body = functools.partial(_region_loss_body, nB=nB, nA=nA, nH=nH, nW=nW, b_blk=b_blk)
    loss = pl.pallas_call(
        body,
        grid=grid,
        in_specs=[
            pl.BlockSpec((b_blk, nC // 2, nH, nW), lambda i: (i, 0, 0, 0)),
            pl.BlockSpec((b_blk, nC // 2, nH, nW), lambda i: (i, 1, 0, 0)),
            pl.BlockSpec((1, b_blk, 4), lambda i: (i, 0, 0)),
            pl.BlockSpec((nA, 2), lambda i: (0, 0)),
        ],
        out_specs=pl.BlockSpec(memory_space=pltpu.SMEM),
        out_shape=jax.ShapeDtypeStruct((1, 1), jnp.float32),
        scratch_shapes=[pltpu.SMEM((3,), jnp.float32)],
    )(output, output, target.reshape(nB // b_blk, b_blk, 4), anchors)
    return loss[0, 0]


# speedup vs baseline: 1.1941x; 1.0412x over previous
import functools

import jax
import jax.numpy as jnp
from jax import lax
from jax.experimental import pallas as pl
from jax.experimental.pallas import tpu as pltpu

_THRESH = 0.6
_OBJECT_SCALE = 5.0
_NOOBJECT_SCALE = 1.0


def _region_loss_body(out_a0, out_a1, tgt_ref, anc_ref, loss_ref, acc_ref, *, nB, nA, nH, nW, b_blk):
    i = pl.program_id(0)

    t = tgt_ref[0]
    gt_x = (t[:, 0:1] * nW)[:, :, None]
    gt_y = (t[:, 1:2] * nH)[:, :, None]
    gt_w = (t[:, 2:3] * nW)[:, :, None]
    gt_h = (t[:, 3:4] * nH)[:, :, None]
    scale = 2.0 - (t[:, 2:3] * t[:, 3:4])[:, :, None]

    aw0 = anc_ref[0, 0]
    ah0 = anc_ref[0, 1]
    aw1 = anc_ref[1, 0]
    ah1 = anc_ref[1, 1]

    gi_f = jnp.floor(gt_x)
    gj_f = jnp.floor(gt_y)
    tx = gt_x - gi_f
    ty = gt_y - gj_f

    inter0 = jnp.minimum(gt_w, aw0) * jnp.minimum(gt_h, ah0)
    union0 = gt_w * gt_h + 1e-16 + aw0 * ah0 - inter0
    inter1 = jnp.minimum(gt_w, aw1) * jnp.minimum(gt_h, ah1)
    union1 = gt_w * gt_h + 1e-16 + aw1 * ah1 - inter1
    best_is_1 = inter1 * union0 > inter0 * union1
    best_aw = jnp.where(best_is_1, aw1, aw0)
    best_ah = jnp.where(best_is_1, ah1, ah0)
    tw = jnp.log(gt_w / best_aw + 1e-16)
    th = jnp.log(gt_h / best_ah + 1e-16)

    ix = lax.broadcasted_iota(jnp.int32, (b_blk, nH, nW), 2).astype(jnp.float32)
    iy = lax.broadcasted_iota(jnp.int32, (b_blk, nH, nW), 1).astype(jnp.float32)

    bx1 = gt_x - gt_w * 0.5
    bx2 = gt_x + gt_w * 0.5
    by1 = gt_y - gt_h * 0.5
    by2 = gt_y + gt_h * 0.5
    barea = gt_w * gt_h

    sum_nm = jnp.float32(0.0)
    sum_cn = jnp.float32(0.0)
    xo = jnp.zeros((b_blk, 1, 1), jnp.float32)
    yo = jnp.zeros((b_blk, 1, 1), jnp.float32)
    wo = jnp.zeros((b_blk, 1, 1), jnp.float32)
    ho = jnp.zeros((b_blk, 1, 1), jnp.float32)
    co = jnp.zeros((b_blk, 1, 1), jnp.float32)

    bf = jnp.bfloat16
    ixb = ix.astype(bf)
    iyb = iy.astype(bf)
    bx1b = bx1.astype(bf)
    bx2b = bx2.astype(bf)
    by1b = by1.astype(bf)
    by2b = by2.astype(bf)
    bareab = barea.astype(bf)
    threshb = bf(_THRESH)
    for a in range(nA):
        aw = aw1 if a == 1 else aw0
        ah = ah1 if a == 1 else ah0
        out_ref = out_a1 if a == 1 else out_a0
        o0 = out_ref[:, 0, :, :]
        o1 = out_ref[:, 1, :, :]
        o2 = out_ref[:, 2, :, :]
        o3 = out_ref[:, 3, :, :]
        o4 = out_ref[:, 4, :, :]
        conf = jax.nn.sigmoid(o4)
        x = jax.nn.sigmoid(o0.astype(bf))
        y = jax.nn.sigmoid(o1.astype(bf))
        pw = jnp.exp(o2.astype(bf)) * aw.astype(bf)
        ph = jnp.exp(o3.astype(bf)) * ah.astype(bf)
        px = x + ixb
        py = y + iyb
        pwh = pw * bf(0.5)
        phh = ph * bf(0.5)
        cw = jnp.minimum(px + pwh, bx2b) - jnp.maximum(px - pwh, bx1b)
        ch = jnp.minimum(py + phh, by2b) - jnp.maximum(py - phh, by1b)
        carea = cw * ch
        uarea = pw * ph + bareab - carea
        hot = (jnp.minimum(cw, ch) > 0) & (carea > threshb * uarea)

        is_best = best_is_1 if a == 1 else ~best_is_1
        onehot = (iy == gj_f) & (ix == gi_f) & is_best
        ohf = jnp.where(onehot, 1.0, 0.0)
        hotf = jnp.where(hot, bf(1), bf(0)).astype(jnp.float32)
        nmf = (1.0 - hotf) * (1.0 - ohf)
        sum_nm = sum_nm + jnp.sum(nmf)
        sum_cn = sum_cn + jnp.sum(conf * conf * nmf)

        xo = xo + jnp.sum(o0 * ohf, axis=(1, 2), keepdims=True)
        yo = yo + jnp.sum(o1 * ohf, axis=(1, 2), keepdims=True)
        wo = wo + jnp.sum(o2 * ohf, axis=(1, 2), keepdims=True)
        ho = ho + jnp.sum(o3 * ohf, axis=(1, 2), keepdims=True)
        co = co + jnp.sum(o4 * ohf, axis=(1, 2), keepdims=True)
    xo = jax.nn.sigmoid(xo)
    yo = jax.nn.sigmoid(yo)
    co = jax.nn.sigmoid(co)

    s2 = scale * scale
    obj = ((xo - tx) ** 2 + (yo - ty) ** 2 + (wo - tw) ** 2 + (ho - th) ** 2) * s2
    obj = obj + _OBJECT_SCALE * (co - 1.0) ** 2
    part_obj = jnp.sum(obj) / jnp.float32(nB)

    @pl.when(i == 0)
    def _init():
        acc_ref[0] = 0.0
        acc_ref[1] = 0.0
        acc_ref[2] = 0.0

    acc_ref[0] = acc_ref[0] + part_obj
    acc_ref[1] = acc_ref[1] + sum_nm
    acc_ref[2] = acc_ref[2] + sum_cn

    @pl.when(i == pl.num_programs(0) - 1)
    def _fin():
        loss_ref[0, 0] = acc_ref[0] + _NOOBJECT_SCALE * acc_ref[2] / acc_ref[1]


def kernel(output, target, anchors):
    nB, nC, nH, nW = output.shape
    nA = anchors.shape[0]
    b_blk = 16
    grid = (nB // b_blk,)
    body = functools.partial(_region_loss_body, nB=nB, nA=nA, nH=nH, nW=nW, b_blk=b_blk)
    loss = pl.pallas_call(
        body,
        grid=grid,
        in_specs=[
            pl.BlockSpec((b_blk, nC // 2, nH, nW), lambda i: (i, 0, 0, 0)),
            pl.BlockSpec((b_blk, nC // 2, nH, nW), lambda i: (i, 1, 0, 0)),
            pl.BlockSpec((1, b_blk, 4), lambda i: (i, 0, 0)),
            pl.BlockSpec((nA, 2), lambda i: (0, 0)),
        ],
        out_specs=pl.BlockSpec(memory_space=pltpu.SMEM),
        out_shape=jax.ShapeDtypeStruct((1, 1), jnp.float32),
        scratch_shapes=[pltpu.SMEM((3,), jnp.float32)],
    )(output, output, target.reshape(nB // b_blk, b_blk, 4), anchors)
    return loss[0, 0]


# speedup vs baseline: 1.2117x; 1.0148x over previous
import functools

import jax
import jax.numpy as jnp
from jax import lax
from jax.experimental import pallas as pl
from jax.experimental.pallas import tpu as pltpu

_THRESH = 0.6
_OBJECT_SCALE = 5.0
_NOOBJECT_SCALE = 1.0


def _region_loss_body(out_a0, out_a1, tgt_ref, anc_ref, loss_ref, acc_ref, *, nB, nA, nH, nW, b_blk):
    i = pl.program_id(0)

    t = tgt_ref[0]
    gt_x = (t[:, 0:1] * nW)[:, :, None]
    gt_y = (t[:, 1:2] * nH)[:, :, None]
    gt_w = (t[:, 2:3] * nW)[:, :, None]
    gt_h = (t[:, 3:4] * nH)[:, :, None]
    scale = 2.0 - (t[:, 2:3] * t[:, 3:4])[:, :, None]

    aw0 = anc_ref[0, 0]
    ah0 = anc_ref[0, 1]
    aw1 = anc_ref[1, 0]
    ah1 = anc_ref[1, 1]

    gi_f = jnp.floor(gt_x)
    gj_f = jnp.floor(gt_y)
    tx = gt_x - gi_f
    ty = gt_y - gj_f

    inter0 = jnp.minimum(gt_w, aw0) * jnp.minimum(gt_h, ah0)
    union0 = gt_w * gt_h + 1e-16 + aw0 * ah0 - inter0
    inter1 = jnp.minimum(gt_w, aw1) * jnp.minimum(gt_h, ah1)
    union1 = gt_w * gt_h + 1e-16 + aw1 * ah1 - inter1
    best_is_1 = inter1 * union0 > inter0 * union1
    best_aw = jnp.where(best_is_1, aw1, aw0)
    best_ah = jnp.where(best_is_1, ah1, ah0)
    tw = jnp.log(gt_w / best_aw + 1e-16)
    th = jnp.log(gt_h / best_ah + 1e-16)

    ix = lax.broadcasted_iota(jnp.int32, (b_blk, nH, nW), 2).astype(jnp.float32)
    iy = lax.broadcasted_iota(jnp.int32, (b_blk, nH, nW), 1).astype(jnp.float32)

    bx1 = gt_x - gt_w * 0.5
    bx2 = gt_x + gt_w * 0.5
    by1 = gt_y - gt_h * 0.5
    by2 = gt_y + gt_h * 0.5
    barea = gt_w * gt_h

    sum_nm = jnp.float32(0.0)
    sum_cn = jnp.float32(0.0)
    xo = jnp.zeros((b_blk, 1, 1), jnp.float32)
    yo = jnp.zeros((b_blk, 1, 1), jnp.float32)
    wo = jnp.zeros((b_blk, 1, 1), jnp.float32)
    ho = jnp.zeros((b_blk, 1, 1), jnp.float32)
    co = jnp.zeros((b_blk, 1, 1), jnp.float32)

    bf = jnp.bfloat16
    ixb = ix.astype(bf)
    iyb = iy.astype(bf)
    bx1b = bx1.astype(bf)
    bx2b = bx2.astype(bf)
    by1b = by1.astype(bf)
    by2b = by2.astype(bf)
    bareab = barea.astype(bf)
    threshb = bf(_THRESH)
    for a in range(nA):
        aw = aw1 if a == 1 else aw0
        ah = ah1 if a == 1 else ah0
        out_ref = out_a1 if a == 1 else out_a0
        o0 = out_ref[:, 0, :, :]
        o1 = out_ref[:, 1, :, :]
        o2 = out_ref[:, 2, :, :]
        o3 = out_ref[:, 3, :, :]
        o4 = out_ref[:, 4, :, :]
        conf = jax.nn.sigmoid(o4.astype(bf))
        x = jax.nn.sigmoid(o0.astype(bf))
        y = jax.nn.sigmoid(o1.astype(bf))
        pw = jnp.exp(o2.astype(bf)) * aw.astype(bf)
        ph = jnp.exp(o3.astype(bf)) * ah.astype(bf)
        px = x + ixb
        py = y + iyb
        pwh = pw * bf(0.5)
        phh = ph * bf(0.5)
        cw = jnp.minimum(px + pwh, bx2b) - jnp.maximum(px - pwh, bx1b)
        ch = jnp.minimum(py + phh, by2b) - jnp.maximum(py - phh, by1b)
        carea = cw * ch
        uarea = pw * ph + bareab - carea
        hot = (jnp.minimum(cw, ch) > 0) & (carea > threshb * uarea)

        is_best = best_is_1 if a == 1 else ~best_is_1
        onehot = (iy == gj_f) & (ix == gi_f) & is_best
        ohf = jnp.where(onehot, 1.0, 0.0)
        hotf = jnp.where(hot, bf(1), bf(0)).astype(jnp.float32)
        nmf = (1.0 - hotf) * (1.0 - ohf)
        sum_nm = sum_nm + jnp.sum(nmf)
        sum_cn = sum_cn + jnp.sum((conf * conf).astype(jnp.float32) * nmf)

        xo = xo + jnp.sum(o0 * ohf, axis=(1, 2), keepdims=True)
        yo = yo + jnp.sum(o1 * ohf, axis=(1, 2), keepdims=True)
        wo = wo + jnp.sum(o2 * ohf, axis=(1, 2), keepdims=True)
        ho = ho + jnp.sum(o3 * ohf, axis=(1, 2), keepdims=True)
        co = co + jnp.sum(o4 * ohf, axis=(1, 2), keepdims=True)
    xo = jax.nn.sigmoid(xo)
    yo = jax.nn.sigmoid(yo)
    co = jax.nn.sigmoid(co)

    s2 = scale * scale
    obj = ((xo - tx) ** 2 + (yo - ty) ** 2 + (wo - tw) ** 2 + (ho - th) ** 2) * s2
    obj = obj + _OBJECT_SCALE * (co - 1.0) ** 2
    part_obj = jnp.sum(obj) / jnp.float32(nB)

    @pl.when(i == 0)
    def _init():
        acc_ref[0] = 0.0
        acc_ref[1] = 0.0
        acc_ref[2] = 0.0

    acc_ref[0] = acc_ref[0] + part_obj
    acc_ref[1] = acc_ref[1] + sum_nm
    acc_ref[2] = acc_ref[2] + sum_cn

    @pl.when(i == pl.num_programs(0) - 1)
    def _fin():
        loss_ref[0, 0] = acc_ref[0] + _NOOBJECT_SCALE * acc_ref[2] / acc_ref[1]


def kernel(output, target, anchors):
    nB, nC, nH, nW = output.shape
    nA = anchors.shape[0]
    b_blk = 16
    grid = (nB // b_blk,)
    body = functools.partial(_region_loss_body, nB=nB, nA=nA, nH=nH, nW=nW, b_blk=b_blk)
    loss = pl.pallas_call(
        body,
        grid=grid,
        in_specs=[
            pl.BlockSpec((b_blk, nC // 2, nH, nW), lambda i: (i, 0, 0, 0)),
            pl.BlockSpec((b_blk, nC // 2, nH, nW), lambda i: (i, 1, 0, 0)),
            pl.BlockSpec((1, b_blk, 4), lambda i: (i, 0, 0)),
            pl.BlockSpec((nA, 2), lambda i: (0, 0)),
        ],
        out_specs=pl.BlockSpec(memory_space=pltpu.SMEM),
        out_shape=jax.ShapeDtypeStruct((1, 1), jnp.float32),
        scratch_shapes=[pltpu.SMEM((3,), jnp.float32)],
    )(output, output, target.reshape(nB // b_blk, b_blk, 4), anchors)
    return loss[0, 0]
